# Initial kernel scaffold; baseline (speedup 1.0000x reference)
import jax
import jax.numpy as jnp
from jax.experimental import pallas as pl


def kernel(x, edge_index, batch, W_enc, b_enc, W_proc, b_proc, W_cls, b_cls):
    raise NotImplementedError("write your pallas kernel here")


# speedup vs baseline: 19.6439x; 19.6439x over previous
import functools
import math

import jax
import jax.numpy as jnp
from jax.experimental import pallas as pl

B = 256
N_NODES = 61
D = 128
C = 10


def _epc_kernel(xb_ref, w_enc_ref, b_enc_ref, w_proc_ref, b_proc_ref,
                w_cls_ref, b_cls_ref, logits_ref, adj_ref):
    f32 = jnp.float32
    x = xb_ref[0]
    h = jax.lax.dot(x, w_enc_ref[...], preferred_element_type=f32) + b_enc_ref[...]
    scores = jax.lax.dot_general(
        h, h, (((1,), (1,)), ((), ())), preferred_element_type=f32)
    scores = scores * f32(1.0 / math.sqrt(D))
    m = jnp.max(scores, axis=1, keepdims=True)
    e = jnp.exp(scores - m)
    rowsum = jnp.sum(e, axis=1, keepdims=True)
    adj = e / rowsum
    adj_ref[0] = adj
    ones = jnp.ones((N_NODES, 1), dtype=f32)
    colsum = jax.lax.dot_general(
        adj, ones, (((0,), (0,)), ((), ())), preferred_element_type=f32)
    nw = jnp.sum(adj, axis=1, keepdims=True) + colsum
    i_idx = jax.lax.broadcasted_iota(jnp.int32, (N_NODES, N_NODES), 0)
    j_idx = jax.lax.broadcasted_iota(jnp.int32, (N_NODES, N_NODES), 1)
    adj_z = jnp.where(i_idx == j_idx, f32(0.0), adj)
    agg = jax.lax.dot_general(
        adj_z, x, (((0,), (0,)), ((), ())), preferred_element_type=f32)
    xp = jnp.maximum(
        jax.lax.dot(agg, w_proc_ref[...], preferred_element_type=f32)
        + b_proc_ref[...], f32(0.0))
    pooled = jax.lax.dot_general(
        nw, xp, (((0,), (0,)), ((), ())), preferred_element_type=f32)
    logits_ref[0] = (jax.lax.dot(pooled, w_cls_ref[...], preferred_element_type=f32)
                     + b_cls_ref[...])


@functools.partial(jax.jit, static_argnums=())
def kernel(x, edge_index, batch, W_enc, b_enc, W_proc, b_proc, W_cls, b_cls):
    del edge_index, batch
    xb = x.reshape(B, N_NODES, D)
    b_enc2 = b_enc.reshape(1, D)
    b_proc2 = b_proc.reshape(1, D)
    b_cls2 = b_cls.reshape(1, C)
    const = lambda b: (0, 0)
    logits3, adj = pl.pallas_call(
        _epc_kernel,
        grid=(B,),
        in_specs=[
            pl.BlockSpec((1, N_NODES, D), lambda b: (b, 0, 0)),
            pl.BlockSpec((D, D), const),
            pl.BlockSpec((1, D), const),
            pl.BlockSpec((D, D), const),
            pl.BlockSpec((1, D), const),
            pl.BlockSpec((D, C), const),
            pl.BlockSpec((1, C), const),
        ],
        out_specs=[
            pl.BlockSpec((1, 1, C), lambda b: (b, 0, 0)),
            pl.BlockSpec((1, N_NODES, N_NODES), lambda b: (b, 0, 0)),
        ],
        out_shape=[
            jax.ShapeDtypeStruct((B, 1, C), jnp.float32),
            jax.ShapeDtypeStruct((B, N_NODES, N_NODES), jnp.float32),
        ],
    )(xb, W_enc, b_enc2, W_proc, b_proc2, W_cls, b_cls2)
    return logits3.reshape(B, C), adj


# speedup vs baseline: 24.2922x; 1.2366x over previous
import functools
import math

import jax
import jax.numpy as jnp
from jax.experimental import pallas as pl

B = 256
N_NODES = 61
D = 128
C = 10
G = 8


def _epc_kernel(xb_ref, w_enc_ref, b_enc_ref, w_proc_ref, b_proc_ref,
                w_cls_ref, b_cls_ref, logits_ref, adj_ref):
    f32 = jnp.float32
    i_idx = jax.lax.broadcasted_iota(jnp.int32, (N_NODES, N_NODES), 0)
    j_idx = jax.lax.broadcasted_iota(jnp.int32, (N_NODES, N_NODES), 1)
    diag = i_idx == j_idx
    ones = jnp.ones((N_NODES, 1), dtype=f32)
    pooled_rows = []
    for g in range(G):
        x = xb_ref[g]
        h = jax.lax.dot(x, w_enc_ref[...], preferred_element_type=f32) + b_enc_ref[...]
        scores = jax.lax.dot_general(
            h, h, (((1,), (1,)), ((), ())), preferred_element_type=f32)
        scores = scores * f32(1.0 / math.sqrt(D))
        m = jnp.max(scores, axis=1, keepdims=True)
        e = jnp.exp(scores - m)
        rowsum = jnp.sum(e, axis=1, keepdims=True)
        adj = e / rowsum
        adj_ref[g] = adj
        colsum = jax.lax.dot_general(
            adj, ones, (((0,), (0,)), ((), ())), preferred_element_type=f32)
        nw = jnp.sum(adj, axis=1, keepdims=True) + colsum
        adj_z = jnp.where(diag, f32(0.0), adj)
        agg = jax.lax.dot_general(
            adj_z, x, (((0,), (0,)), ((), ())), preferred_element_type=f32)
        xp = jnp.maximum(
            jax.lax.dot(agg, w_proc_ref[...], preferred_element_type=f32)
            + b_proc_ref[...], f32(0.0))
        pooled_rows.append(jax.lax.dot_general(
            nw, xp, (((0,), (0,)), ((), ())), preferred_element_type=f32))
    pooled = jnp.concatenate(pooled_rows, axis=0)
    logits_ref[...] = (jax.lax.dot(pooled, w_cls_ref[...],
                                   preferred_element_type=f32) + b_cls_ref[...])


@functools.partial(jax.jit, static_argnums=())
def kernel(x, edge_index, batch, W_enc, b_enc, W_proc, b_proc, W_cls, b_cls):
    del edge_index, batch
    xb = x.reshape(B, N_NODES, D)
    b_enc2 = b_enc.reshape(1, D)
    b_proc2 = b_proc.reshape(1, D)
    b_cls2 = b_cls.reshape(1, C)
    const = lambda b: (0, 0)
    logits, adj = pl.pallas_call(
        _epc_kernel,
        grid=(B // G,),
        in_specs=[
            pl.BlockSpec((G, N_NODES, D), lambda b: (b, 0, 0)),
            pl.BlockSpec((D, D), const),
            pl.BlockSpec((1, D), const),
            pl.BlockSpec((D, D), const),
            pl.BlockSpec((1, D), const),
            pl.BlockSpec((D, C), const),
            pl.BlockSpec((1, C), const),
        ],
        out_specs=[
            pl.BlockSpec((G, C), lambda b: (b, 0)),
            pl.BlockSpec((G, N_NODES, N_NODES), lambda b: (b, 0, 0)),
        ],
        out_shape=[
            jax.ShapeDtypeStruct((B, C), jnp.float32),
            jax.ShapeDtypeStruct((B, N_NODES, N_NODES), jnp.float32),
        ],
    )(xb, W_enc, b_enc2, W_proc, b_proc2, W_cls, b_cls2)
    return logits, adj


# speedup vs baseline: 100.6667x; 4.1440x over previous
import functools
import math

import jax
import jax.numpy as jnp
from jax.experimental import pallas as pl

B = 256
N_NODES = 61
D = 128
C = 10
G = 8


def _epc_kernel(xb_ref, w_enc_ref, b_enc_ref, w_proc_ref, b_proc_ref,
                w_cls_ref, b_cls_ref, logits_ref, adj_ref):
    f32 = jnp.float32
    i_idx = jax.lax.broadcasted_iota(jnp.int32, (N_NODES, N_NODES), 0)
    j_idx = jax.lax.broadcasted_iota(jnp.int32, (N_NODES, N_NODES), 1)
    diag = i_idx == j_idx
    inv_sqrt_d = f32(1.0 / math.sqrt(D))
    w_enc = w_enc_ref[...]
    b_enc = b_enc_ref[...]
    w_proc = w_proc_ref[...]
    b_proc = b_proc_ref[...]

    xs = [xb_ref[g] for g in range(G)]
    hs = [jax.lax.dot(x, w_enc, preferred_element_type=f32) + b_enc for x in xs]
    ss = [jax.lax.dot_general(h, h, (((1,), (1,)), ((), ())),
                              preferred_element_type=f32) * inv_sqrt_d
          for h in hs]
    adjTs = []
    for s in ss:
        m = jnp.max(s, axis=0, keepdims=True)
        eT = jnp.exp(s - m)
        ssum = jnp.sum(eT, axis=0, keepdims=True)
        adjTs.append(eT / ssum)
    aggs = [jax.lax.dot(jnp.where(diag, f32(0.0), adjT), x,
                        preferred_element_type=f32)
            for adjT, x in zip(adjTs, xs)]
    xps = [jnp.maximum(jax.lax.dot(agg, w_proc, preferred_element_type=f32)
                       + b_proc, f32(0.0))
           for agg in aggs]
    adjs = [adjT.T for adjT in adjTs]
    for g in range(G):
        adj_ref[g] = adjs[g]
    nws = [jnp.sum(adjT, axis=0, keepdims=True)
           + jnp.sum(adj, axis=0, keepdims=True)
           for adjT, adj in zip(adjTs, adjs)]
    pooled = jnp.concatenate(
        [jax.lax.dot(nw, xp, preferred_element_type=f32)
         for nw, xp in zip(nws, xps)], axis=0)
    logits_ref[...] = (jax.lax.dot(pooled, w_cls_ref[...],
                                   preferred_element_type=f32) + b_cls_ref[...])


@functools.partial(jax.jit, static_argnums=())
def kernel(x, edge_index, batch, W_enc, b_enc, W_proc, b_proc, W_cls, b_cls):
    del edge_index, batch
    xb = x.reshape(B, N_NODES, D)
    b_enc2 = b_enc.reshape(1, D)
    b_proc2 = b_proc.reshape(1, D)
    b_cls2 = b_cls.reshape(1, C)
    const = lambda b: (0, 0)
    logits, adj = pl.pallas_call(
        _epc_kernel,
        grid=(B // G,),
        in_specs=[
            pl.BlockSpec((G, N_NODES, D), lambda b: (b, 0, 0)),
            pl.BlockSpec((D, D), const),
            pl.BlockSpec((1, D), const),
            pl.BlockSpec((D, D), const),
            pl.BlockSpec((1, D), const),
            pl.BlockSpec((D, C), const),
            pl.BlockSpec((1, C), const),
        ],
        out_specs=[
            pl.BlockSpec((G, C), lambda b: (b, 0)),
            pl.BlockSpec((G, N_NODES, N_NODES), lambda b: (b, 0, 0)),
        ],
        out_shape=[
            jax.ShapeDtypeStruct((B, C), jnp.float32),
            jax.ShapeDtypeStruct((B, N_NODES, N_NODES), jnp.float32),
        ],
    )(xb, W_enc, b_enc2, W_proc, b_proc2, W_cls, b_cls2)
    return logits, adj


# speedup vs baseline: 132.5979x; 1.3172x over previous
import functools
import math

import jax
import jax.numpy as jnp
from jax.experimental import pallas as pl

B = 256
N_NODES = 61
D = 128
C = 10
G = 16


def _epc_kernel(xb_ref, w_enc_ref, b_enc_ref, w_proc_ref, b_proc_ref,
                w_cls_ref, b_cls_ref, logits_ref, adj_ref):
    f32 = jnp.float32
    i_idx = jax.lax.broadcasted_iota(jnp.int32, (N_NODES, N_NODES), 0)
    j_idx = jax.lax.broadcasted_iota(jnp.int32, (N_NODES, N_NODES), 1)
    diag = i_idx == j_idx
    inv_sqrt_d = f32(1.0 / math.sqrt(D))
    w_enc = w_enc_ref[...]
    b_enc = b_enc_ref[...]
    w_proc = w_proc_ref[...]
    b_proc = b_proc_ref[...]

    xs = [xb_ref[g] for g in range(G)]
    hs = [jax.lax.dot(x, w_enc, preferred_element_type=f32) + b_enc for x in xs]
    ss = [jax.lax.dot_general(h, h, (((1,), (1,)), ((), ())),
                              preferred_element_type=f32) * inv_sqrt_d
          for h in hs]
    adjTs = []
    for s in ss:
        m = jnp.max(s, axis=0, keepdims=True)
        eT = jnp.exp(s - m)
        ssum = jnp.sum(eT, axis=0, keepdims=True)
        adjTs.append(eT / ssum)
    aggs = [jax.lax.dot(jnp.where(diag, f32(0.0), adjT), x,
                        preferred_element_type=f32)
            for adjT, x in zip(adjTs, xs)]
    xps = [jnp.maximum(jax.lax.dot(agg, w_proc, preferred_element_type=f32)
                       + b_proc, f32(0.0))
           for agg in aggs]
    adjs = [adjT.T for adjT in adjTs]
    for g in range(G):
        adj_ref[g] = adjs[g]
    nws = [jnp.sum(adjT, axis=0, keepdims=True)
           + jnp.sum(adj, axis=0, keepdims=True)
           for adjT, adj in zip(adjTs, adjs)]
    pooled = jnp.concatenate(
        [jax.lax.dot(nw, xp, preferred_element_type=f32)
         for nw, xp in zip(nws, xps)], axis=0)
    logits_ref[...] = (jax.lax.dot(pooled, w_cls_ref[...],
                                   preferred_element_type=f32) + b_cls_ref[...])


@functools.partial(jax.jit, static_argnums=())
def kernel(x, edge_index, batch, W_enc, b_enc, W_proc, b_proc, W_cls, b_cls):
    del edge_index, batch
    xb = x.reshape(B, N_NODES, D)
    b_enc2 = b_enc.reshape(1, D)
    b_proc2 = b_proc.reshape(1, D)
    b_cls2 = b_cls.reshape(1, C)
    const = lambda b: (0, 0)
    logits, adj = pl.pallas_call(
        _epc_kernel,
        grid=(B // G,),
        in_specs=[
            pl.BlockSpec((G, N_NODES, D), lambda b: (b, 0, 0)),
            pl.BlockSpec((D, D), const),
            pl.BlockSpec((1, D), const),
            pl.BlockSpec((D, D), const),
            pl.BlockSpec((1, D), const),
            pl.BlockSpec((D, C), const),
            pl.BlockSpec((1, C), const),
        ],
        out_specs=[
            pl.BlockSpec((G, C), lambda b: (b, 0)),
            pl.BlockSpec((G, N_NODES, N_NODES), lambda b: (b, 0, 0)),
        ],
        out_shape=[
            jax.ShapeDtypeStruct((B, C), jnp.float32),
            jax.ShapeDtypeStruct((B, N_NODES, N_NODES), jnp.float32),
        ],
    )(xb, W_enc, b_enc2, W_proc, b_proc2, W_cls, b_cls2)
    return logits, adj


# speedup vs baseline: 148.5971x; 1.1207x over previous
import functools
import math

import jax
import jax.numpy as jnp
from jax.experimental import pallas as pl

B = 256
N_NODES = 61
D = 128
C = 10
G = 32


def _epc_kernel(xb_ref, w_enc_ref, b_enc_ref, w_proc_ref, b_proc_ref,
                w_cls_ref, b_cls_ref, logits_ref, adj_ref):
    f32 = jnp.float32
    i_idx = jax.lax.broadcasted_iota(jnp.int32, (N_NODES, N_NODES), 0)
    j_idx = jax.lax.broadcasted_iota(jnp.int32, (N_NODES, N_NODES), 1)
    diag = i_idx == j_idx
    inv_sqrt_d = f32(1.0 / math.sqrt(D))
    w_enc = w_enc_ref[...]
    b_enc = b_enc_ref[...]
    w_proc = w_proc_ref[...]
    b_proc = b_proc_ref[...]

    xs = [xb_ref[g] for g in range(G)]
    hs = [jax.lax.dot(x, w_enc, preferred_element_type=f32) + b_enc for x in xs]
    ss = [jax.lax.dot_general(h, h, (((1,), (1,)), ((), ())),
                              preferred_element_type=f32) * inv_sqrt_d
          for h in hs]
    adjTs = []
    for s in ss:
        m = jnp.max(s, axis=0, keepdims=True)
        eT = jnp.exp(s - m)
        ssum = jnp.sum(eT, axis=0, keepdims=True)
        adjTs.append(eT / ssum)
    aggs = [jax.lax.dot(jnp.where(diag, f32(0.0), adjT), x,
                        preferred_element_type=f32)
            for adjT, x in zip(adjTs, xs)]
    xps = [jnp.maximum(jax.lax.dot(agg, w_proc, preferred_element_type=f32)
                       + b_proc, f32(0.0))
           for agg in aggs]
    adjs = [adjT.T for adjT in adjTs]
    for g in range(G):
        adj_ref[g] = adjs[g]
    nws = [jnp.sum(adjT, axis=0, keepdims=True)
           + jnp.sum(adj, axis=0, keepdims=True)
           for adjT, adj in zip(adjTs, adjs)]
    pooled = jnp.concatenate(
        [jax.lax.dot(nw, xp, preferred_element_type=f32)
         for nw, xp in zip(nws, xps)], axis=0)
    logits_ref[...] = (jax.lax.dot(pooled, w_cls_ref[...],
                                   preferred_element_type=f32) + b_cls_ref[...])


@functools.partial(jax.jit, static_argnums=())
def kernel(x, edge_index, batch, W_enc, b_enc, W_proc, b_proc, W_cls, b_cls):
    del edge_index, batch
    xb = x.reshape(B, N_NODES, D)
    b_enc2 = b_enc.reshape(1, D)
    b_proc2 = b_proc.reshape(1, D)
    b_cls2 = b_cls.reshape(1, C)
    const = lambda b: (0, 0)
    logits, adj = pl.pallas_call(
        _epc_kernel,
        grid=(B // G,),
        in_specs=[
            pl.BlockSpec((G, N_NODES, D), lambda b: (b, 0, 0)),
            pl.BlockSpec((D, D), const),
            pl.BlockSpec((1, D), const),
            pl.BlockSpec((D, D), const),
            pl.BlockSpec((1, D), const),
            pl.BlockSpec((D, C), const),
            pl.BlockSpec((1, C), const),
        ],
        out_specs=[
            pl.BlockSpec((G, C), lambda b: (b, 0)),
            pl.BlockSpec((G, N_NODES, N_NODES), lambda b: (b, 0, 0)),
        ],
        out_shape=[
            jax.ShapeDtypeStruct((B, C), jnp.float32),
            jax.ShapeDtypeStruct((B, N_NODES, N_NODES), jnp.float32),
        ],
    )(xb, W_enc, b_enc2, W_proc, b_proc2, W_cls, b_cls2)
    return logits, adj


# speedup vs baseline: 150.7029x; 1.0142x over previous
import functools
import math

import jax
import jax.numpy as jnp
from jax.experimental import pallas as pl

B = 256
N_NODES = 61
D = 128
C = 10
G = 64


def _epc_kernel(xb_ref, w_enc_ref, b_enc_ref, w_proc_ref, b_proc_ref,
                w_cls_ref, b_cls_ref, logits_ref, adj_ref):
    f32 = jnp.float32
    i_idx = jax.lax.broadcasted_iota(jnp.int32, (N_NODES, N_NODES), 0)
    j_idx = jax.lax.broadcasted_iota(jnp.int32, (N_NODES, N_NODES), 1)
    diag = i_idx == j_idx
    inv_sqrt_d = f32(1.0 / math.sqrt(D))
    w_enc = w_enc_ref[...]
    b_enc = b_enc_ref[...]
    w_proc = w_proc_ref[...]
    b_proc = b_proc_ref[...]

    xs = [xb_ref[g] for g in range(G)]
    hs = [jax.lax.dot(x, w_enc, preferred_element_type=f32) + b_enc for x in xs]
    ss = [jax.lax.dot_general(h, h, (((1,), (1,)), ((), ())),
                              preferred_element_type=f32) * inv_sqrt_d
          for h in hs]
    adjTs = []
    for s in ss:
        m = jnp.max(s, axis=0, keepdims=True)
        eT = jnp.exp(s - m)
        ssum = jnp.sum(eT, axis=0, keepdims=True)
        adjTs.append(eT / ssum)
    aggs = [jax.lax.dot(jnp.where(diag, f32(0.0), adjT), x,
                        preferred_element_type=f32)
            for adjT, x in zip(adjTs, xs)]
    xps = [jnp.maximum(jax.lax.dot(agg, w_proc, preferred_element_type=f32)
                       + b_proc, f32(0.0))
           for agg in aggs]
    adjs = [adjT.T for adjT in adjTs]
    for g in range(G):
        adj_ref[g] = adjs[g]
    nws = [jnp.sum(adjT, axis=0, keepdims=True)
           + jnp.sum(adj, axis=0, keepdims=True)
           for adjT, adj in zip(adjTs, adjs)]
    pooled = jnp.concatenate(
        [jax.lax.dot(nw, xp, preferred_element_type=f32)
         for nw, xp in zip(nws, xps)], axis=0)
    logits_ref[...] = (jax.lax.dot(pooled, w_cls_ref[...],
                                   preferred_element_type=f32) + b_cls_ref[...])


@functools.partial(jax.jit, static_argnums=())
def kernel(x, edge_index, batch, W_enc, b_enc, W_proc, b_proc, W_cls, b_cls):
    del edge_index, batch
    xb = x.reshape(B, N_NODES, D)
    b_enc2 = b_enc.reshape(1, D)
    b_proc2 = b_proc.reshape(1, D)
    b_cls2 = b_cls.reshape(1, C)
    const = lambda b: (0, 0)
    logits, adj = pl.pallas_call(
        _epc_kernel,
        grid=(B // G,),
        in_specs=[
            pl.BlockSpec((G, N_NODES, D), lambda b: (b, 0, 0)),
            pl.BlockSpec((D, D), const),
            pl.BlockSpec((1, D), const),
            pl.BlockSpec((D, D), const),
            pl.BlockSpec((1, D), const),
            pl.BlockSpec((D, C), const),
            pl.BlockSpec((1, C), const),
        ],
        out_specs=[
            pl.BlockSpec((G, C), lambda b: (b, 0)),
            pl.BlockSpec((G, N_NODES, N_NODES), lambda b: (b, 0, 0)),
        ],
        out_shape=[
            jax.ShapeDtypeStruct((B, C), jnp.float32),
            jax.ShapeDtypeStruct((B, N_NODES, N_NODES), jnp.float32),
        ],
    )(xb, W_enc, b_enc2, W_proc, b_proc2, W_cls, b_cls2)
    return logits, adj


# speedup vs baseline: 151.1408x; 1.0029x over previous
import functools
import math

import jax
import jax.numpy as jnp
from jax.experimental import pallas as pl
from jax.experimental.pallas import tpu as pltpu

B = 256
N_NODES = 61
D = 128
C = 10
G = 64


def _epc_kernel(xb_ref, w_enc_ref, b_enc_ref, w_proc_ref, b_proc_ref,
                w_cls_ref, b_cls_ref, logits_ref, adj_ref):
    f32 = jnp.float32
    i_idx = jax.lax.broadcasted_iota(jnp.int32, (N_NODES, N_NODES), 0)
    j_idx = jax.lax.broadcasted_iota(jnp.int32, (N_NODES, N_NODES), 1)
    diag = i_idx == j_idx
    inv_sqrt_d = f32(1.0 / math.sqrt(D))
    w_enc = w_enc_ref[...]
    b_enc = b_enc_ref[...]
    w_proc = w_proc_ref[...]
    b_proc = b_proc_ref[...]

    xs = [xb_ref[g] for g in range(G)]
    hs = [jax.lax.dot(x, w_enc, preferred_element_type=f32) + b_enc for x in xs]
    ss = [jax.lax.dot_general(h, h, (((1,), (1,)), ((), ())),
                              preferred_element_type=f32) * inv_sqrt_d
          for h in hs]
    adjTs = []
    for s in ss:
        m = jnp.max(s, axis=0, keepdims=True)
        eT = jnp.exp(s - m)
        ssum = jnp.sum(eT, axis=0, keepdims=True)
        adjTs.append(eT / ssum)
    aggs = [jax.lax.dot(jnp.where(diag, f32(0.0), adjT), x,
                        preferred_element_type=f32)
            for adjT, x in zip(adjTs, xs)]
    xps = [jnp.maximum(jax.lax.dot(agg, w_proc, preferred_element_type=f32)
                       + b_proc, f32(0.0))
           for agg in aggs]
    adjs = [adjT.T for adjT in adjTs]
    for g in range(G):
        adj_ref[g] = adjs[g]
    nws = [jnp.sum(adjT, axis=0, keepdims=True)
           + jnp.sum(adj, axis=0, keepdims=True)
           for adjT, adj in zip(adjTs, adjs)]
    pooled = jnp.concatenate(
        [jax.lax.dot(nw, xp, preferred_element_type=f32)
         for nw, xp in zip(nws, xps)], axis=0)
    logits_ref[...] = (jax.lax.dot(pooled, w_cls_ref[...],
                                   preferred_element_type=f32) + b_cls_ref[...])


@functools.partial(jax.jit, static_argnums=())
def kernel(x, edge_index, batch, W_enc, b_enc, W_proc, b_proc, W_cls, b_cls):
    del edge_index, batch
    xb = x.reshape(B, N_NODES, D)
    b_enc2 = b_enc.reshape(1, D)
    b_proc2 = b_proc.reshape(1, D)
    b_cls2 = b_cls.reshape(1, C)
    const = lambda b: (0, 0)
    logits, adj = pl.pallas_call(
        _epc_kernel,
        grid=(B // G,),
        in_specs=[
            pl.BlockSpec((G, N_NODES, D), lambda b: (b, 0, 0)),
            pl.BlockSpec((D, D), const),
            pl.BlockSpec((1, D), const),
            pl.BlockSpec((D, D), const),
            pl.BlockSpec((1, D), const),
            pl.BlockSpec((D, C), const),
            pl.BlockSpec((1, C), const),
        ],
        out_specs=[
            pl.BlockSpec((G, C), lambda b: (b, 0)),
            pl.BlockSpec((G, N_NODES, N_NODES), lambda b: (b, 0, 0)),
        ],
        out_shape=[
            jax.ShapeDtypeStruct((B, C), jnp.float32),
            jax.ShapeDtypeStruct((B, N_NODES, N_NODES), jnp.float32),
        ],
        compiler_params=pltpu.CompilerParams(
            dimension_semantics=("parallel",)),
    )(xb, W_enc, b_enc2, W_proc, b_proc2, W_cls, b_cls2)
    return logits, adj
